# baseline (speedup 1.0000x reference)
import functools

import jax
import jax.numpy as jnp
from jax import lax
from jax.experimental import pallas as pl
from jax.experimental.pallas import tpu as pltpu
from jax.experimental.pallas import tpu_sc as plsc

B, E, H = 8, 16384, 256
MR = MC = MT = 2048
N_R = B * MR * 6
N_CT = B * (MC + MT) * 4
NW = 32
CHUNK = 128
PW_R = N_R // NW
PW_CT = N_CT // NW
HW = H // 2


def _sc_gather(table, idx_r, idx_ct):
    mesh = plsc.VectorSubcoreMesh(core_axis_name="c", subcore_axis_name="s")

    @functools.partial(
        pl.kernel,
        mesh=mesh,
        out_type=(
            jax.ShapeDtypeStruct((N_R, HW), jnp.int32),
            jax.ShapeDtypeStruct((N_CT, HW), jnp.int32),
        ),
        scratch_types=[
            pltpu.VMEM((PW_R,), jnp.int32),
            pltpu.VMEM((PW_CT,), jnp.int32),
            pltpu.VMEM((CHUNK, HW), jnp.int32),
            pltpu.VMEM((CHUNK, HW), jnp.int32),
            pltpu.SemaphoreType.DMA,
            pltpu.SemaphoreType.DMA,
        ],
    )
    def gather_kernel(table_hbm, idxr_hbm, idxct_hbm, outr_hbm, outct_hbm,
                      idxr_v, idxct_v, buf0, buf1, sem0, sem1):
        wid = lax.axis_index("s") * 2 + lax.axis_index("c")

        def run(idx_hbm, idx_v, out_hbm, per_w):
            base = wid * per_w
            nchunk = per_w // CHUNK
            pltpu.sync_copy(idx_hbm.at[pl.ds(base, per_w)], idx_v)

            def issue(c, buf, sem):
                pltpu.async_copy(
                    table_hbm.at[idx_v.at[pl.ds(c * CHUNK, CHUNK)]], buf, sem
                )

            def drain(c, buf, sem):
                pltpu.make_async_copy(
                    table_hbm.at[idx_v.at[pl.ds(c * CHUNK, CHUNK)]], buf, sem
                ).wait()
                pltpu.sync_copy(buf, out_hbm.at[pl.ds(base + c * CHUNK, CHUNK)])

            issue(0, buf0, sem0)

            def body(p, carry):
                c0 = p * 2
                issue(c0 + 1, buf1, sem1)
                drain(c0, buf0, sem0)

                @pl.when(c0 + 2 < nchunk)
                def _issue_next():
                    issue(c0 + 2, buf0, sem0)

                drain(c0 + 1, buf1, sem1)
                return carry

            lax.fori_loop(0, nchunk // 2, body, 0)

        run(idxr_hbm, idxr_v, outr_hbm, PW_R)
        run(idxct_hbm, idxct_v, outct_hbm, PW_CT)

    return gather_kernel(table, idx_r, idx_ct)


def _mlp_body(x_ref, w1_ref, b1_ref, w2_ref, b2_ref,
              wa1_ref, ba1_ref, wa2_ref, ba2_ref, wa3_ref, ba3_ref,
              wa4_ref, ba4_ref, out_ref):
    f32 = jnp.float32
    bf16 = jnp.bfloat16

    def lin(h, w_ref, b_ref):
        return jnp.dot(h, w_ref[...], preferred_element_type=f32) + b_ref[...]

    x = x_ref[...]
    h = jnp.maximum(lin(x, w1_ref, b1_ref), 0.0).astype(bf16)
    m = lin(h, w2_ref, b2_ref).astype(bf16)
    h = jnp.maximum(lin(m, wa1_ref, ba1_ref), 0.0).astype(bf16)
    h = jnp.maximum(lin(h, wa2_ref, ba2_ref), 0.0).astype(bf16)
    h = jnp.maximum(lin(h, wa3_ref, ba3_ref), 0.0).astype(bf16)
    out_ref[...] = lin(h, wa4_ref, ba4_ref)


def _mlp_stack(x, w1, b1, w2, b2, wa1, ba1, wa2, ba2, wa3, ba3, wa4, ba4, row_block):
    n, k = x.shape
    grid = (n // row_block,)
    fixed = lambda i: (0, 0)
    out = pl.pallas_call(
        _mlp_body,
        grid=grid,
        in_specs=[
            pl.BlockSpec((row_block, k), lambda i: (i, 0)),
            pl.BlockSpec((k, H), fixed),
            pl.BlockSpec((1, H), fixed),
            pl.BlockSpec((H, H), fixed),
            pl.BlockSpec((1, H), fixed),
            pl.BlockSpec((H, H), fixed),
            pl.BlockSpec((1, H), fixed),
            pl.BlockSpec((H, H), fixed),
            pl.BlockSpec((1, H), fixed),
            pl.BlockSpec((H, H), fixed),
            pl.BlockSpec((1, H), fixed),
            pl.BlockSpec((H, 1), fixed),
            pl.BlockSpec((1, 1), fixed),
        ],
        out_specs=pl.BlockSpec((row_block, 1), lambda i: (i, 0)),
        out_shape=jax.ShapeDtypeStruct((n, 1), jnp.float32),
    )(x, w1, b1, w2, b2, wa1, ba1, wa2, ba2, wa3, ba3, wa4, ba4)
    return out


def kernel(e_emb, reloc_idx, cross_idx, twoopt_idx,
           Wr1, br1, Wr2, br2,
           Wc1, bc1, Wc2, bc2,
           Wa1, ba1, Wa2, ba2, Wa3, ba3, Wa4, ba4):
    bf16 = jnp.bfloat16
    offs = (jnp.arange(B, dtype=jnp.int32) * E)[:, None, None]
    ridx = (reloc_idx.astype(jnp.int32) + offs).reshape(-1)
    cidx = (cross_idx.astype(jnp.int32) + offs).reshape(-1)
    tidx = (twoopt_idx.astype(jnp.int32) + offs).reshape(-1)
    idx_ct = jnp.concatenate([cidx, tidx])

    table = e_emb.reshape(B * E, H).astype(bf16)
    table_i32 = lax.bitcast_convert_type(table.reshape(B * E, HW, 2), jnp.int32)
    g_r, g_ct = _sc_gather(table_i32, ridx, idx_ct)

    tobf = lambda g: lax.bitcast_convert_type(g, bf16)
    xr = tobf(g_r).reshape(B * MR, 6 * H)
    xct = tobf(g_ct).reshape(B * (MC + MT), 4 * H)

    r1 = lambda v: v.reshape(1, -1)
    wb = lambda w: w.astype(bf16)
    logits_r = _mlp_stack(xr, wb(Wr1), r1(br1), wb(Wr2), r1(br2),
                          wb(Wa1), r1(ba1), wb(Wa2), r1(ba2), wb(Wa3), r1(ba3),
                          wb(Wa4), r1(ba4), row_block=2048)
    logits_ct = _mlp_stack(xct, wb(Wc1), r1(bc1), wb(Wc2), r1(bc2),
                           wb(Wa1), r1(ba1), wb(Wa2), r1(ba2), wb(Wa3), r1(ba3),
                           wb(Wa4), r1(ba4), row_block=2048)

    lr = logits_r.reshape(B, MR)
    lc = logits_ct[: B * MC].reshape(B, MC)
    lt = logits_ct[B * MC:].reshape(B, MT)
    return jnp.concatenate([lr, lc, lt], axis=1)


# speedup vs baseline: 36.9835x; 36.9835x over previous
import functools

import jax
import jax.numpy as jnp
from jax import lax
from jax.experimental import pallas as pl
from jax.experimental.pallas import tpu as pltpu
from jax.experimental.pallas import tpu_sc as plsc

B, E, H = 8, 16384, 256
MR = MC = MT = 2048
N_R = B * MR * 6
N_CT = B * (MC + MT) * 4
NW = 32
CHUNK = 128
PW_R = N_R // NW
PW_CT = N_CT // NW
def _sc_gather(table, idx_r, idx_ct):
    mesh = plsc.VectorSubcoreMesh(core_axis_name="c", subcore_axis_name="s")

    @functools.partial(
        pl.kernel,
        mesh=mesh,
        out_type=(
            jax.ShapeDtypeStruct((N_R, H), jnp.float32),
            jax.ShapeDtypeStruct((N_CT, H), jnp.float32),
        ),
        scratch_types=[
            pltpu.VMEM((PW_R,), jnp.int32),
            pltpu.VMEM((PW_CT,), jnp.int32),
            pltpu.VMEM((CHUNK, H), jnp.float32),
            pltpu.VMEM((CHUNK, H), jnp.float32),
            pltpu.SemaphoreType.DMA,
            pltpu.SemaphoreType.DMA,
        ],
    )
    def gather_kernel(table_hbm, idxr_hbm, idxct_hbm, outr_hbm, outct_hbm,
                      idxr_v, idxct_v, buf0, buf1, sem0, sem1):
        wid = lax.axis_index("s") * 2 + lax.axis_index("c")

        def run(idx_hbm, idx_v, out_hbm, per_w):
            base = wid * per_w
            nchunk = per_w // CHUNK
            pltpu.sync_copy(idx_hbm.at[pl.ds(base, per_w)], idx_v)

            def issue(c, buf, sem):
                pltpu.async_copy(
                    table_hbm.at[idx_v.at[pl.ds(c * CHUNK, CHUNK)]], buf, sem
                )

            def drain(c, buf, sem):
                pltpu.make_async_copy(
                    table_hbm.at[idx_v.at[pl.ds(c * CHUNK, CHUNK)]], buf, sem
                ).wait()
                pltpu.sync_copy(buf, out_hbm.at[pl.ds(base + c * CHUNK, CHUNK)])

            issue(0, buf0, sem0)

            def body(p, carry):
                c0 = p * 2
                issue(c0 + 1, buf1, sem1)
                drain(c0, buf0, sem0)

                @pl.when(c0 + 2 < nchunk)
                def _issue_next():
                    issue(c0 + 2, buf0, sem0)

                drain(c0 + 1, buf1, sem1)
                return carry

            lax.fori_loop(0, nchunk // 2, body, 0)

        run(idxr_hbm, idxr_v, outr_hbm, PW_R)
        run(idxct_hbm, idxct_v, outct_hbm, PW_CT)

    return gather_kernel(table, idx_r, idx_ct)


def _mlp_body(x_ref, w1_ref, b1_ref, w2_ref, b2_ref,
              wa1_ref, ba1_ref, wa2_ref, ba2_ref, wa3_ref, ba3_ref,
              wa4_ref, ba4_ref, out_ref):
    f32 = jnp.float32
    bf16 = jnp.bfloat16

    def lin(h, w_ref, b_ref):
        return jnp.dot(h, w_ref[...], preferred_element_type=f32) + b_ref[...]

    x = x_ref[...].astype(bf16)
    h = jnp.maximum(lin(x, w1_ref, b1_ref), 0.0).astype(bf16)
    m = lin(h, w2_ref, b2_ref).astype(bf16)
    h = jnp.maximum(lin(m, wa1_ref, ba1_ref), 0.0).astype(bf16)
    h = jnp.maximum(lin(h, wa2_ref, ba2_ref), 0.0).astype(bf16)
    h = jnp.maximum(lin(h, wa3_ref, ba3_ref), 0.0).astype(bf16)
    out_ref[...] = lin(h, wa4_ref, ba4_ref)


def _mlp_stack(x, w1, b1, w2, b2, wa1, ba1, wa2, ba2, wa3, ba3, wa4, ba4, row_block):
    n, k = x.shape
    grid = (n // row_block,)
    fixed = lambda i: (0, 0)
    out = pl.pallas_call(
        _mlp_body,
        grid=grid,
        in_specs=[
            pl.BlockSpec((row_block, k), lambda i: (i, 0)),
            pl.BlockSpec((k, H), fixed),
            pl.BlockSpec((1, H), fixed),
            pl.BlockSpec((H, H), fixed),
            pl.BlockSpec((1, H), fixed),
            pl.BlockSpec((H, H), fixed),
            pl.BlockSpec((1, H), fixed),
            pl.BlockSpec((H, H), fixed),
            pl.BlockSpec((1, H), fixed),
            pl.BlockSpec((H, H), fixed),
            pl.BlockSpec((1, H), fixed),
            pl.BlockSpec((H, 1), fixed),
            pl.BlockSpec((1, 1), fixed),
        ],
        out_specs=pl.BlockSpec((row_block, 1), lambda i: (i, 0)),
        out_shape=jax.ShapeDtypeStruct((n, 1), jnp.float32),
    )(x, w1, b1, w2, b2, wa1, ba1, wa2, ba2, wa3, ba3, wa4, ba4)
    return out


def kernel(e_emb, reloc_idx, cross_idx, twoopt_idx,
           Wr1, br1, Wr2, br2,
           Wc1, bc1, Wc2, bc2,
           Wa1, ba1, Wa2, ba2, Wa3, ba3, Wa4, ba4):
    bf16 = jnp.bfloat16
    offs = (jnp.arange(B, dtype=jnp.int32) * E)[:, None, None]
    ridx = (reloc_idx.astype(jnp.int32) + offs).reshape(-1)
    cidx = (cross_idx.astype(jnp.int32) + offs).reshape(-1)
    tidx = (twoopt_idx.astype(jnp.int32) + offs).reshape(-1)
    idx_ct = jnp.concatenate([cidx, tidx])

    table = e_emb.reshape(B * E, H)
    g_r, g_ct = _sc_gather(table, ridx, idx_ct)

    xr = g_r.reshape(B * MR, 6 * H)
    xct = g_ct.reshape(B * (MC + MT), 4 * H)

    r1 = lambda v: v.reshape(1, -1)
    wb = lambda w: w.astype(bf16)
    logits_r = _mlp_stack(xr, wb(Wr1), r1(br1), wb(Wr2), r1(br2),
                          wb(Wa1), r1(ba1), wb(Wa2), r1(ba2), wb(Wa3), r1(ba3),
                          wb(Wa4), r1(ba4), row_block=2048)
    logits_ct = _mlp_stack(xct, wb(Wc1), r1(bc1), wb(Wc2), r1(bc2),
                           wb(Wa1), r1(ba1), wb(Wa2), r1(ba2), wb(Wa3), r1(ba3),
                           wb(Wa4), r1(ba4), row_block=2048)

    lr = logits_r.reshape(B, MR)
    lc = logits_ct[: B * MC].reshape(B, MC)
    lt = logits_ct[B * MC:].reshape(B, MT)
    return jnp.concatenate([lr, lc, lt], axis=1)


# speedup vs baseline: 66.7023x; 1.8036x over previous
import functools

import jax
import jax.numpy as jnp
from jax import lax
from jax.experimental import pallas as pl
from jax.experimental.pallas import tpu as pltpu
from jax.experimental.pallas import tpu_sc as plsc

B, E, H = 8, 16384, 256
MR = MC = MT = 2048
K_R, K_CT = 6, 4
M_RF = B * MR
M_CTF = B * (MC + MT)
NW = 32
CHUNK = 128
PW_RF = M_RF // NW
PW_CTF = M_CTF // NW


def _sc_gather(table, idx_r, idx_ct):
    mesh = plsc.VectorSubcoreMesh(core_axis_name="c", subcore_axis_name="s")
    out_type = tuple(
        jax.ShapeDtypeStruct((M_RF, H), jnp.float32) for _ in range(K_R)
    ) + tuple(
        jax.ShapeDtypeStruct((M_CTF, H), jnp.float32) for _ in range(K_CT)
    )

    @functools.partial(
        pl.kernel,
        mesh=mesh,
        out_type=out_type,
        scratch_types=[
            pltpu.VMEM((PW_CTF,), jnp.int32),
            pltpu.VMEM((CHUNK, H), jnp.float32),
            pltpu.VMEM((CHUNK, H), jnp.float32),
            pltpu.SemaphoreType.DMA,
            pltpu.SemaphoreType.DMA,
        ],
    )
    def gather_kernel(table_hbm, idxr_hbm, idxct_hbm, *refs):
        out_refs = refs[:K_R + K_CT]
        idx_v, buf0, buf1, sem0, sem1 = refs[K_R + K_CT:]
        wid = lax.axis_index("s") * 2 + lax.axis_index("c")

        def run(idx_hbm, idx_base, out_hbm, per_w):
            base = wid * per_w
            nchunk = per_w // CHUNK
            pltpu.sync_copy(
                idx_hbm.at[pl.ds(idx_base + base, per_w)],
                idx_v.at[pl.ds(0, per_w)],
            )

            def issue(c, buf, sem):
                pltpu.async_copy(
                    table_hbm.at[idx_v.at[pl.ds(c * CHUNK, CHUNK)]], buf, sem
                )

            def drain(c, buf, sem):
                pltpu.make_async_copy(
                    table_hbm.at[idx_v.at[pl.ds(c * CHUNK, CHUNK)]], buf, sem
                ).wait()
                pltpu.sync_copy(buf, out_hbm.at[pl.ds(base + c * CHUNK, CHUNK)])

            issue(0, buf0, sem0)

            def body(p, carry):
                c0 = p * 2
                issue(c0 + 1, buf1, sem1)
                drain(c0, buf0, sem0)

                @pl.when(c0 + 2 < nchunk)
                def _issue_next():
                    issue(c0 + 2, buf0, sem0)

                drain(c0 + 1, buf1, sem1)
                return carry

            lax.fori_loop(0, nchunk // 2, body, 0)

        for j in range(K_R):
            run(idxr_hbm, j * M_RF, out_refs[j], PW_RF)
        for j in range(K_CT):
            run(idxct_hbm, j * M_CTF, out_refs[K_R + j], PW_CTF)

    return gather_kernel(table, idx_r, idx_ct)


def _mk_mlp_body(k):
    def body(*refs):
        x_refs = refs[:k]
        (w1_ref, b1_ref, w2_ref, b2_ref,
         wa1_ref, ba1_ref, wa2_ref, ba2_ref, wa3_ref, ba3_ref,
         wa4_ref, ba4_ref, out_ref) = refs[k:]
        f32 = jnp.float32
        bf16 = jnp.bfloat16

        def lin(h, w_ref, b_ref):
            return jnp.dot(h, w_ref[...], preferred_element_type=f32) + b_ref[...]

        s = b1_ref[...].astype(f32)
        for j in range(k):
            xj = x_refs[j][...].astype(bf16)
            s = s + jnp.dot(xj, w1_ref[j * H:(j + 1) * H, :],
                            preferred_element_type=f32)
        h = jnp.maximum(s, 0.0).astype(bf16)
        m = lin(h, w2_ref, b2_ref).astype(bf16)
        h = jnp.maximum(lin(m, wa1_ref, ba1_ref), 0.0).astype(bf16)
        h = jnp.maximum(lin(h, wa2_ref, ba2_ref), 0.0).astype(bf16)
        h = jnp.maximum(lin(h, wa3_ref, ba3_ref), 0.0).astype(bf16)
        out_ref[...] = lin(h, wa4_ref, ba4_ref)
    return body


def _mlp_stack(xs, w1, b1, w2, b2, wa1, ba1, wa2, ba2, wa3, ba3, wa4, ba4,
               row_block):
    k = len(xs)
    n = xs[0].shape[0]
    grid = (n // row_block,)
    fixed = lambda i: (0, 0)
    out = pl.pallas_call(
        _mk_mlp_body(k),
        grid=grid,
        in_specs=[pl.BlockSpec((row_block, H), lambda i: (i, 0))] * k + [
            pl.BlockSpec((k * H, H), fixed),
            pl.BlockSpec((1, H), fixed),
            pl.BlockSpec((H, H), fixed),
            pl.BlockSpec((1, H), fixed),
            pl.BlockSpec((H, H), fixed),
            pl.BlockSpec((1, H), fixed),
            pl.BlockSpec((H, H), fixed),
            pl.BlockSpec((1, H), fixed),
            pl.BlockSpec((H, H), fixed),
            pl.BlockSpec((1, H), fixed),
            pl.BlockSpec((H, 1), fixed),
            pl.BlockSpec((1, 1), fixed),
        ],
        out_specs=pl.BlockSpec((row_block, 1), lambda i: (i, 0)),
        out_shape=jax.ShapeDtypeStruct((n, 1), jnp.float32),
    )(*xs, w1, b1, w2, b2, wa1, ba1, wa2, ba2, wa3, ba3, wa4, ba4)
    return out


def kernel(e_emb, reloc_idx, cross_idx, twoopt_idx,
           Wr1, br1, Wr2, br2,
           Wc1, bc1, Wc2, bc2,
           Wa1, ba1, Wa2, ba2, Wa3, ba3, Wa4, ba4):
    bf16 = jnp.bfloat16
    offs = (jnp.arange(B, dtype=jnp.int32) * E)[:, None, None]
    ridx = jnp.transpose(reloc_idx.astype(jnp.int32) + offs, (2, 0, 1)).reshape(-1)
    cidx = jnp.transpose(cross_idx.astype(jnp.int32) + offs, (2, 0, 1)).reshape(K_CT, -1)
    tidx = jnp.transpose(twoopt_idx.astype(jnp.int32) + offs, (2, 0, 1)).reshape(K_CT, -1)
    ctidx = jnp.concatenate([cidx, tidx], axis=1).reshape(-1)

    table = e_emb.reshape(B * E, H)
    gs = _sc_gather(table, ridx, ctidx)
    xs_r = gs[:K_R]
    xs_ct = gs[K_R:]

    r1 = lambda v: v.reshape(1, -1)
    wb = lambda w: w.astype(bf16)
    logits_r = _mlp_stack(xs_r, wb(Wr1), r1(br1), wb(Wr2), r1(br2),
                          wb(Wa1), r1(ba1), wb(Wa2), r1(ba2), wb(Wa3), r1(ba3),
                          wb(Wa4), r1(ba4), row_block=2048)
    logits_ct = _mlp_stack(xs_ct, wb(Wc1), r1(bc1), wb(Wc2), r1(bc2),
                           wb(Wa1), r1(ba1), wb(Wa2), r1(ba2), wb(Wa3), r1(ba3),
                           wb(Wa4), r1(ba4), row_block=2048)

    lr = logits_r.reshape(B, MR)
    lc = logits_ct[: B * MC].reshape(B, MC)
    lt = logits_ct[B * MC:].reshape(B, MT)
    return jnp.concatenate([lr, lc, lt], axis=1)


# speedup vs baseline: 69.9110x; 1.0481x over previous
import functools

import jax
import jax.numpy as jnp
from jax import lax
from jax.experimental import pallas as pl
from jax.experimental.pallas import tpu as pltpu
from jax.experimental.pallas import tpu_sc as plsc

B, E, H = 8, 16384, 256
MR = MC = MT = 2048
K_R, K_CT = 6, 4
M_RF = B * MR
M_CTF = B * (MC + MT)
NW = 32
CHUNK = 128
PW_RF = M_RF // NW
PW_CTF = M_CTF // NW
HW = H // 2


def _pack_body(x_ref, out_ref):
    x = x_ref[...]
    lo = lax.bitcast_convert_type(x[:, :HW].astype(jnp.bfloat16), jnp.uint16)
    hi = lax.bitcast_convert_type(x[:, HW:].astype(jnp.bfloat16), jnp.uint16)
    w = lo.astype(jnp.uint32) | (hi.astype(jnp.uint32) << 16)
    out_ref[...] = lax.bitcast_convert_type(w, jnp.int32)


def _pack_table(table):
    n = table.shape[0]
    rb = 4096
    return pl.pallas_call(
        _pack_body,
        grid=(n // rb,),
        in_specs=[pl.BlockSpec((rb, H), lambda i: (i, 0))],
        out_specs=pl.BlockSpec((rb, HW), lambda i: (i, 0)),
        out_shape=jax.ShapeDtypeStruct((n, HW), jnp.int32),
    )(table)


def _sc_gather(table, idx_r, idx_ct):
    mesh = plsc.VectorSubcoreMesh(core_axis_name="c", subcore_axis_name="s")
    out_type = tuple(
        jax.ShapeDtypeStruct((M_RF, HW), jnp.int32) for _ in range(K_R)
    ) + tuple(
        jax.ShapeDtypeStruct((M_CTF, HW), jnp.int32) for _ in range(K_CT)
    )

    @functools.partial(
        pl.kernel,
        mesh=mesh,
        out_type=out_type,
        scratch_types=[
            pltpu.VMEM((PW_CTF,), jnp.int32),
            pltpu.VMEM((CHUNK, HW), jnp.int32),
            pltpu.VMEM((CHUNK, HW), jnp.int32),
            pltpu.SemaphoreType.DMA,
            pltpu.SemaphoreType.DMA,
        ],
    )
    def gather_kernel(table_hbm, idxr_hbm, idxct_hbm, *refs):
        out_refs = refs[:K_R + K_CT]
        idx_v, buf0, buf1, sem0, sem1 = refs[K_R + K_CT:]
        wid = lax.axis_index("s") * 2 + lax.axis_index("c")

        def run(idx_hbm, idx_base, out_hbm, per_w):
            base = wid * per_w
            nchunk = per_w // CHUNK
            pltpu.sync_copy(
                idx_hbm.at[pl.ds(idx_base + base, per_w)],
                idx_v.at[pl.ds(0, per_w)],
            )

            def issue(c, buf, sem):
                pltpu.async_copy(
                    table_hbm.at[idx_v.at[pl.ds(c * CHUNK, CHUNK)]], buf, sem
                )

            def drain(c, buf, sem):
                pltpu.make_async_copy(
                    table_hbm.at[idx_v.at[pl.ds(c * CHUNK, CHUNK)]], buf, sem
                ).wait()
                pltpu.sync_copy(buf, out_hbm.at[pl.ds(base + c * CHUNK, CHUNK)])

            issue(0, buf0, sem0)

            def body(p, carry):
                c0 = p * 2
                issue(c0 + 1, buf1, sem1)
                drain(c0, buf0, sem0)

                @pl.when(c0 + 2 < nchunk)
                def _issue_next():
                    issue(c0 + 2, buf0, sem0)

                drain(c0 + 1, buf1, sem1)
                return carry

            lax.fori_loop(0, nchunk // 2, body, 0)

        for j in range(K_R):
            run(idxr_hbm, j * M_RF, out_refs[j], PW_RF)
        for j in range(K_CT):
            run(idxct_hbm, j * M_CTF, out_refs[K_R + j], PW_CTF)

    return gather_kernel(table, idx_r, idx_ct)


def _mk_mlp_body(k):
    def body(*refs):
        x_refs = refs[:k]
        (w1_ref, b1_ref, w2_ref, b2_ref,
         wa1_ref, ba1_ref, wa2_ref, ba2_ref, wa3_ref, ba3_ref,
         wa4_ref, ba4_ref, out_ref) = refs[k:]
        f32 = jnp.float32
        bf16 = jnp.bfloat16

        def lin(h, w_ref, b_ref):
            return jnp.dot(h, w_ref[...], preferred_element_type=f32) + b_ref[...]

        s = b1_ref[...].astype(f32)
        for j in range(k):
            w = x_refs[j][...]
            lo = lax.bitcast_convert_type(w << 16, f32)
            hi = lax.bitcast_convert_type(w & jnp.int32(-65536), f32)
            xj = jnp.concatenate([lo, hi], axis=1).astype(bf16)
            s = s + jnp.dot(xj, w1_ref[j * H:(j + 1) * H, :],
                            preferred_element_type=f32)
        h = jnp.maximum(s, 0.0).astype(bf16)
        m = lin(h, w2_ref, b2_ref).astype(bf16)
        h = jnp.maximum(lin(m, wa1_ref, ba1_ref), 0.0).astype(bf16)
        h = jnp.maximum(lin(h, wa2_ref, ba2_ref), 0.0).astype(bf16)
        h = jnp.maximum(lin(h, wa3_ref, ba3_ref), 0.0).astype(bf16)
        out_ref[...] = lin(h, wa4_ref, ba4_ref)
    return body


def _mlp_stack(xs, w1, b1, w2, b2, wa1, ba1, wa2, ba2, wa3, ba3, wa4, ba4,
               row_block):
    k = len(xs)
    n = xs[0].shape[0]
    grid = (n // row_block,)
    fixed = lambda i: (0, 0)
    out = pl.pallas_call(
        _mk_mlp_body(k),
        grid=grid,
        in_specs=[pl.BlockSpec((row_block, HW), lambda i: (i, 0))] * k + [
            pl.BlockSpec((k * H, H), fixed),
            pl.BlockSpec((1, H), fixed),
            pl.BlockSpec((H, H), fixed),
            pl.BlockSpec((1, H), fixed),
            pl.BlockSpec((H, H), fixed),
            pl.BlockSpec((1, H), fixed),
            pl.BlockSpec((H, H), fixed),
            pl.BlockSpec((1, H), fixed),
            pl.BlockSpec((H, H), fixed),
            pl.BlockSpec((1, H), fixed),
            pl.BlockSpec((H, 1), fixed),
            pl.BlockSpec((1, 1), fixed),
        ],
        out_specs=pl.BlockSpec((row_block, 1), lambda i: (i, 0)),
        out_shape=jax.ShapeDtypeStruct((n, 1), jnp.float32),
    )(*xs, w1, b1, w2, b2, wa1, ba1, wa2, ba2, wa3, ba3, wa4, ba4)
    return out


def kernel(e_emb, reloc_idx, cross_idx, twoopt_idx,
           Wr1, br1, Wr2, br2,
           Wc1, bc1, Wc2, bc2,
           Wa1, ba1, Wa2, ba2, Wa3, ba3, Wa4, ba4):
    bf16 = jnp.bfloat16
    offs = (jnp.arange(B, dtype=jnp.int32) * E)[:, None, None]
    ridx = jnp.transpose(reloc_idx.astype(jnp.int32) + offs, (2, 0, 1)).reshape(-1)
    cidx = jnp.transpose(cross_idx.astype(jnp.int32) + offs, (2, 0, 1)).reshape(K_CT, -1)
    tidx = jnp.transpose(twoopt_idx.astype(jnp.int32) + offs, (2, 0, 1)).reshape(K_CT, -1)
    ctidx = jnp.concatenate([cidx, tidx], axis=1).reshape(-1)

    table = _pack_table(e_emb.reshape(B * E, H))
    gs = _sc_gather(table, ridx, ctidx)
    xs_r = gs[:K_R]
    xs_ct = gs[K_R:]

    r1 = lambda v: v.reshape(1, -1)
    wb = lambda w: w.astype(bf16)
    logits_r = _mlp_stack(xs_r, wb(Wr1), r1(br1), wb(Wr2), r1(br2),
                          wb(Wa1), r1(ba1), wb(Wa2), r1(ba2), wb(Wa3), r1(ba3),
                          wb(Wa4), r1(ba4), row_block=2048)
    logits_ct = _mlp_stack(xs_ct, wb(Wc1), r1(bc1), wb(Wc2), r1(bc2),
                           wb(Wa1), r1(ba1), wb(Wa2), r1(ba2), wb(Wa3), r1(ba3),
                           wb(Wa4), r1(ba4), row_block=2048)

    lr = logits_r.reshape(B, MR)
    lc = logits_ct[: B * MC].reshape(B, MC)
    lt = logits_ct[B * MC:].reshape(B, MT)
    return jnp.concatenate([lr, lc, lt], axis=1)


# speedup vs baseline: 79.0269x; 1.1304x over previous
import functools

import jax
import jax.numpy as jnp
from jax import lax
from jax.experimental import pallas as pl
from jax.experimental.pallas import tpu as pltpu
from jax.experimental.pallas import tpu_sc as plsc

B, E, H = 8, 16384, 256
MR = MC = MT = 2048
K_R, K_CT = 6, 4
M_RF = B * MR
M_CTF = B * (MC + MT)
NW = 32
CHUNK = 128
PW_RF = M_RF // NW
PW_CTF = M_CTF // NW
HW = H // 2


def _pack_body(x_ref, out_ref):
    x = x_ref[...]
    lo = lax.bitcast_convert_type(x[:, :HW].astype(jnp.bfloat16), jnp.uint16)
    hi = lax.bitcast_convert_type(x[:, HW:].astype(jnp.bfloat16), jnp.uint16)
    w = lo.astype(jnp.uint32) | (hi.astype(jnp.uint32) << 16)
    out_ref[...] = lax.bitcast_convert_type(w, jnp.int32)


def _pack_table(table):
    n = table.shape[0]
    rb = 4096
    return pl.pallas_call(
        _pack_body,
        grid=(n // rb,),
        in_specs=[pl.BlockSpec((rb, H), lambda i: (i, 0))],
        out_specs=pl.BlockSpec((rb, HW), lambda i: (i, 0)),
        out_shape=jax.ShapeDtypeStruct((n, HW), jnp.int32),
    )(table)


def _sc_gather(table, idx, k, m_rows):
    mesh = plsc.VectorSubcoreMesh(core_axis_name="c", subcore_axis_name="s")
    per_w = m_rows // NW

    @functools.partial(
        pl.kernel,
        mesh=mesh,
        out_type=tuple(
            jax.ShapeDtypeStruct((m_rows, HW), jnp.int32) for _ in range(k)
        ),
        scratch_types=[
            pltpu.VMEM((per_w,), jnp.int32),
            pltpu.VMEM((CHUNK, HW), jnp.int32),
            pltpu.VMEM((CHUNK, HW), jnp.int32),
            pltpu.SemaphoreType.DMA,
            pltpu.SemaphoreType.DMA,
        ],
    )
    def gather_kernel(table_hbm, idx_hbm, *refs):
        out_refs = refs[:k]
        idx_v, buf0, buf1, sem0, sem1 = refs[k:]
        wid = lax.axis_index("s") * 2 + lax.axis_index("c")

        def run(idx_base, out_hbm):
            base = wid * per_w
            nchunk = per_w // CHUNK
            pltpu.sync_copy(
                idx_hbm.at[pl.ds(idx_base + base, per_w)],
                idx_v.at[pl.ds(0, per_w)],
            )

            def issue(c, buf, sem):
                pltpu.async_copy(
                    table_hbm.at[idx_v.at[pl.ds(c * CHUNK, CHUNK)]], buf, sem
                )

            def drain(c, buf, sem):
                pltpu.make_async_copy(
                    table_hbm.at[idx_v.at[pl.ds(c * CHUNK, CHUNK)]], buf, sem
                ).wait()
                pltpu.sync_copy(buf, out_hbm.at[pl.ds(base + c * CHUNK, CHUNK)])

            issue(0, buf0, sem0)

            def body(p, carry):
                c0 = p * 2
                issue(c0 + 1, buf1, sem1)
                drain(c0, buf0, sem0)

                @pl.when(c0 + 2 < nchunk)
                def _issue_next():
                    issue(c0 + 2, buf0, sem0)

                drain(c0 + 1, buf1, sem1)
                return carry

            lax.fori_loop(0, nchunk // 2, body, 0)

        for j in range(k):
            run(j * m_rows, out_refs[j])

    return gather_kernel(table, idx)


def _mk_mlp_body(k):
    def body(*refs):
        x_refs = refs[:k]
        (w1_ref, b1_ref, w2_ref, b2_ref,
         wa1_ref, ba1_ref, wa2_ref, ba2_ref, wa3_ref, ba3_ref,
         wa4_ref, ba4_ref, out_ref) = refs[k:]
        f32 = jnp.float32
        bf16 = jnp.bfloat16

        def lin(h, w_ref, b_ref):
            return jnp.dot(h, w_ref[...], preferred_element_type=f32) + b_ref[...]

        s = b1_ref[...].astype(f32)
        for j in range(k):
            w = x_refs[j][...]
            lo = lax.bitcast_convert_type(w << 16, f32)
            hi = lax.bitcast_convert_type(w & jnp.int32(-65536), f32)
            xj = jnp.concatenate([lo, hi], axis=1).astype(bf16)
            s = s + jnp.dot(xj, w1_ref[j * H:(j + 1) * H, :],
                            preferred_element_type=f32)
        h = jnp.maximum(s, 0.0).astype(bf16)
        m = lin(h, w2_ref, b2_ref).astype(bf16)
        h = jnp.maximum(lin(m, wa1_ref, ba1_ref), 0.0).astype(bf16)
        h = jnp.maximum(lin(h, wa2_ref, ba2_ref), 0.0).astype(bf16)
        h = jnp.maximum(lin(h, wa3_ref, ba3_ref), 0.0).astype(bf16)
        out_ref[...] = lin(h, wa4_ref, ba4_ref)
    return body


def _mlp_stack(xs, w1, b1, w2, b2, wa1, ba1, wa2, ba2, wa3, ba3, wa4, ba4,
               row_block):
    k = len(xs)
    n = xs[0].shape[0]
    grid = (n // row_block,)
    fixed = lambda i: (0, 0)
    out = pl.pallas_call(
        _mk_mlp_body(k),
        grid=grid,
        in_specs=[pl.BlockSpec((row_block, HW), lambda i: (i, 0))] * k + [
            pl.BlockSpec((k * H, H), fixed),
            pl.BlockSpec((1, H), fixed),
            pl.BlockSpec((H, H), fixed),
            pl.BlockSpec((1, H), fixed),
            pl.BlockSpec((H, H), fixed),
            pl.BlockSpec((1, H), fixed),
            pl.BlockSpec((H, H), fixed),
            pl.BlockSpec((1, H), fixed),
            pl.BlockSpec((H, H), fixed),
            pl.BlockSpec((1, H), fixed),
            pl.BlockSpec((H, 1), fixed),
            pl.BlockSpec((1, 1), fixed),
        ],
        out_specs=pl.BlockSpec((row_block, 1), lambda i: (i, 0)),
        out_shape=jax.ShapeDtypeStruct((n, 1), jnp.float32),
    )(*xs, w1, b1, w2, b2, wa1, ba1, wa2, ba2, wa3, ba3, wa4, ba4)
    return out


def kernel(e_emb, reloc_idx, cross_idx, twoopt_idx,
           Wr1, br1, Wr2, br2,
           Wc1, bc1, Wc2, bc2,
           Wa1, ba1, Wa2, ba2, Wa3, ba3, Wa4, ba4):
    bf16 = jnp.bfloat16
    offs = (jnp.arange(B, dtype=jnp.int32) * E)[:, None, None]
    ridx = jnp.transpose(reloc_idx.astype(jnp.int32) + offs, (2, 0, 1)).reshape(-1)
    cidx = jnp.transpose(cross_idx.astype(jnp.int32) + offs, (2, 0, 1)).reshape(K_CT, -1)
    tidx = jnp.transpose(twoopt_idx.astype(jnp.int32) + offs, (2, 0, 1)).reshape(K_CT, -1)
    ctidx = jnp.concatenate([cidx, tidx], axis=1).reshape(-1)

    table = _pack_table(e_emb.reshape(B * E, H))
    xs_r = _sc_gather(table, ridx, K_R, M_RF)
    xs_ct = _sc_gather(table, ctidx, K_CT, M_CTF)

    r1 = lambda v: v.reshape(1, -1)
    wb = lambda w: w.astype(bf16)
    logits_r = _mlp_stack(xs_r, wb(Wr1), r1(br1), wb(Wr2), r1(br2),
                          wb(Wa1), r1(ba1), wb(Wa2), r1(ba2), wb(Wa3), r1(ba3),
                          wb(Wa4), r1(ba4), row_block=2048)
    logits_ct = _mlp_stack(xs_ct, wb(Wc1), r1(bc1), wb(Wc2), r1(bc2),
                           wb(Wa1), r1(ba1), wb(Wa2), r1(ba2), wb(Wa3), r1(ba3),
                           wb(Wa4), r1(ba4), row_block=2048)

    lr = logits_r.reshape(B, MR)
    lc = logits_ct[: B * MC].reshape(B, MC)
    lt = logits_ct[B * MC:].reshape(B, MT)
    return jnp.concatenate([lr, lc, lt], axis=1)
